# baseline (speedup 1.0000x reference)
import functools

import jax
import jax.numpy as jnp
from jax import lax
from jax.experimental import pallas as pl
from jax.experimental.pallas import tpu as pltpu
from jax.experimental.pallas import tpu_sc as plsc

DIM = 2048
N_CHANNELS = 32
TABLE = 2 * DIM - 1
NSHIFT = 128
TW = 3968
N_Q = 8
N_K = 8
N_V = 16
CHUNK = 32
NBLOCKS = DIM // NSHIFT


def _emit_rows(dst_hbm, c_local, table_hbm, c_global, chunk, sem):
    for q in range(NSHIFT // CHUNK):
        pltpu.sync_copy(
            table_hbm.at[c_global, pl.ds(q * CHUNK, CHUNK), :], chunk
        )
        for m in range(NBLOCKS):
            a = (NBLOCKS - 1 - m) * NSHIFT
            pltpu.make_async_copy(
                chunk.at[:, pl.ds(a, DIM)],
                dst_hbm.at[c_local, pl.ds(m * NSHIFT + q * CHUNK, CHUNK)],
                sem,
            ).start()
        for m in range(NBLOCKS):
            pltpu.make_async_copy(
                chunk.at[:, pl.ds(0, DIM)],
                dst_hbm.at[c_local, pl.ds(0, CHUNK)],
                sem,
            ).wait()


@functools.partial(
    pl.kernel,
    out_type=(
        jax.ShapeDtypeStruct((N_Q, DIM, DIM), jnp.float32),
        jax.ShapeDtypeStruct((N_K, DIM, DIM), jnp.float32),
        jax.ShapeDtypeStruct((N_V, DIM, DIM), jnp.float32),
    ),
    mesh=plsc.VectorSubcoreMesh(core_axis_name="c", subcore_axis_name="s"),
    scratch_types=[
        pltpu.VMEM((CHUNK, TW), jnp.float32),
        pltpu.SemaphoreType.DMA,
    ],
)
def _sc_expand(table_hbm, q_hbm, k_hbm, v_hbm, chunk, sem):
    wid = lax.axis_index("s") * 2 + lax.axis_index("c")

    @pl.when(wid < N_Q)
    def _():
        _emit_rows(q_hbm, wid, table_hbm, wid, chunk, sem)

    @pl.when((wid >= N_Q) & (wid < N_Q + N_K))
    def _():
        _emit_rows(k_hbm, wid - N_Q, table_hbm, wid, chunk, sem)

    @pl.when(wid >= N_Q + N_K)
    def _():
        _emit_rows(v_hbm, wid - (N_Q + N_K), table_hbm, wid, chunk, sem)


def kernel(relative, flatten_index):
    del flatten_index
    rev = relative[:, ::-1]
    revp = jnp.pad(rev, ((0, 0), (0, NSHIFT)))
    table = jnp.stack(
        [revp[:, NSHIFT - 1 - r : NSHIFT - 1 - r + TW] for r in range(NSHIFT)],
        axis=1,
    )
    return _sc_expand(table)


# speedup vs baseline: 5.5086x; 5.5086x over previous
import functools

import jax
import jax.numpy as jnp
from jax import lax
from jax.experimental import pallas as pl
from jax.experimental.pallas import tpu as pltpu
from jax.experimental.pallas import tpu_sc as plsc

DIM = 2048
N_CHANNELS = 32
TABLE = 2 * DIM - 1
NSHIFT = 128
TW = 3968
N_Q = 8
N_K = 8
N_V = 16
CHUNK = 32
NBLOCKS = DIM // NSHIFT


def _emit_rows(dst_hbm, c_local, table_hbm, c_global, chunk, sem):
    for q in range(NSHIFT // CHUNK):
        pltpu.sync_copy(
            table_hbm.at[c_global, pl.ds(q * CHUNK, CHUNK), :], chunk
        )
        for m in range(NBLOCKS):
            a = (NBLOCKS - 1 - m) * NSHIFT
            pltpu.make_async_copy(
                chunk.at[:, pl.ds(a, DIM)],
                dst_hbm.at[c_local, pl.ds(m * NSHIFT + q * CHUNK, CHUNK)],
                sem,
            ).start()
        for m in range(NBLOCKS):
            pltpu.make_async_copy(
                chunk.at[:, pl.ds(0, DIM)],
                dst_hbm.at[c_local, pl.ds(0, CHUNK)],
                sem,
            ).wait()


@functools.partial(
    pl.kernel,
    out_type=(
        jax.ShapeDtypeStruct((N_Q, DIM, DIM), jnp.float32),
        jax.ShapeDtypeStruct((N_K, DIM, DIM), jnp.float32),
        jax.ShapeDtypeStruct((N_V, DIM, DIM), jnp.float32),
    ),
    mesh=plsc.VectorSubcoreMesh(core_axis_name="c", subcore_axis_name="s"),
    scratch_types=[
        pltpu.VMEM((CHUNK, TW), jnp.float32),
        pltpu.SemaphoreType.DMA,
    ],
)
def _sc_expand(table_hbm, q_hbm, k_hbm, v_hbm, chunk, sem):
    wid = lax.axis_index("s") * 2 + lax.axis_index("c")

    @pl.when(wid < N_Q)
    def _():
        _emit_rows(q_hbm, wid, table_hbm, wid, chunk, sem)

    @pl.when((wid >= N_Q) & (wid < N_Q + N_K))
    def _():
        _emit_rows(k_hbm, wid - N_Q, table_hbm, wid, chunk, sem)

    @pl.when(wid >= N_Q + N_K)
    def _():
        _emit_rows(v_hbm, wid - (N_Q + N_K), table_hbm, wid, chunk, sem)


def kernel(relative, flatten_index):
    del flatten_index
    rev = relative[:, ::-1]
    flat = jnp.broadcast_to(rev[:, None, :], (N_CHANNELS, NSHIFT, TABLE))
    flat = flat.reshape(N_CHANNELS, NSHIFT * TABLE)
    table = flat[:, NSHIFT - 1 : NSHIFT - 1 + NSHIFT * (TABLE - 1)]
    table = table.reshape(N_CHANNELS, NSHIFT, TABLE - 1)[:, :, :TW]
    return _sc_expand(table)
